# baseline (speedup 1.0000x reference)
import functools
import math

import jax
import jax.numpy as jnp
from jax import lax
from jax.experimental import pallas as pl
from jax.experimental.pallas import tpu as pltpu
from jax.experimental.pallas import tpu_sc as plsc

_NC = 2
_NS = 16
_NW = _NC * _NS
_LANES = 16


@functools.lru_cache(maxsize=None)
def _build(b, d, chunk):
    bpw = b // _NW
    nchunks = bpw // chunk
    scale = jnp.float32(math.sqrt(d))
    mesh = plsc.VectorSubcoreMesh(core_axis_name="c", subcore_axis_name="s")

    @functools.partial(
        pl.kernel,
        mesh=mesh,
        out_type=jax.ShapeDtypeStruct((b, d), jnp.float32),
        scratch_types=[
            pltpu.VMEM((bpw,), jnp.int32),
            pltpu.VMEM((chunk, d), jnp.float32),
            pltpu.SemaphoreType.DMA,
        ],
    )
    def emb(idx_hbm, table_hbm, out_hbm, idx_v, rows_v, sem):
        wid = lax.axis_index("s") * _NC + lax.axis_index("c")
        base = wid * bpw
        pltpu.sync_copy(idx_hbm.at[pl.ds(base, bpw)], idx_v)
        for c in range(nchunks):
            off = c * chunk
            pltpu.async_copy(
                table_hbm.at[idx_v.at[pl.ds(off, chunk)]], rows_v, sem
            ).wait()

            def body(i, carry):
                for j in range(d // _LANES):
                    sl = pl.ds(j * _LANES, _LANES)
                    rows_v[i, sl] = rows_v[i, sl] * scale
                return carry

            lax.fori_loop(0, chunk, body, 0)
            pltpu.sync_copy(rows_v, out_hbm.at[pl.ds(base + off, chunk)])

    return emb


def kernel(token_idx, table):
    d = table.shape[1]
    idx = token_idx.reshape(-1).astype(jnp.int32)
    b = idx.shape[0]
    out = _build(b, d, 80)(idx, table)
    return out.reshape(*token_idx.shape, d)


# speedup vs baseline: 1.1270x; 1.1270x over previous
import functools
import math

import jax
import jax.numpy as jnp
from jax import lax
from jax.experimental import pallas as pl
from jax.experimental.pallas import tpu as pltpu
from jax.experimental.pallas import tpu_sc as plsc

_NC = 2
_NS = 16
_NW = _NC * _NS
_LANES = 16


@functools.lru_cache(maxsize=None)
def _build(b, d, chunk):
    bpw = b // _NW
    nchunks = bpw // chunk
    scale = jnp.float32(math.sqrt(d))
    mesh = plsc.VectorSubcoreMesh(core_axis_name="c", subcore_axis_name="s")
    row_buf = pltpu.VMEM((chunk, d), jnp.float32)

    @functools.partial(
        pl.kernel,
        mesh=mesh,
        out_type=jax.ShapeDtypeStruct((b, d), jnp.float32),
        scratch_types=[
            pltpu.VMEM((bpw,), jnp.int32),
            row_buf, row_buf, row_buf, row_buf,
            pltpu.SemaphoreType.DMA,
            pltpu.SemaphoreType.DMA,
            pltpu.SemaphoreType.DMA,
            pltpu.SemaphoreType.DMA,
        ],
    )
    def emb(idx_hbm, table_hbm, out_hbm, idx_v, ib0, ib1, ob0, ob1,
            si0, si1, so0, so1):
        ib = (ib0, ib1)
        ob = (ob0, ob1)
        si = (si0, si1)
        so = (so0, so1)
        wid = lax.axis_index("s") * _NC + lax.axis_index("c")
        base = wid * bpw
        pltpu.sync_copy(idx_hbm.at[pl.ds(base, bpw)], idx_v)

        def gather(c):
            return pltpu.async_copy(
                table_hbm.at[idx_v.at[pl.ds(c * chunk, chunk)]],
                ib[c % 2], si[c % 2])

        def scale_chunk(c):
            src, dst = ib[c % 2], ob[c % 2]

            def body(i, carry):
                for j in range(d // _LANES):
                    sl = pl.ds(j * _LANES, _LANES)
                    dst[i, sl] = src[i, sl] * scale
                return carry

            lax.fori_loop(0, chunk, body, 0)

        def put(c):
            return pltpu.async_copy(
                ob[c % 2], out_hbm.at[pl.ds(base + c * chunk, chunk)],
                so[c % 2])

        inc = {0: gather(0)}
        if nchunks > 1:
            inc[1] = gather(1)
        outc = {}
        for c in range(nchunks):
            inc[c].wait()
            if c >= 2:
                outc[c - 2].wait()
            scale_chunk(c)
            outc[c] = put(c)
            if c + 2 < nchunks:
                inc[c + 2] = gather(c + 2)
        outc[nchunks - 2].wait()
        outc[nchunks - 1].wait()

    return emb


def kernel(token_idx, table):
    d = table.shape[1]
    idx = token_idx.reshape(-1).astype(jnp.int32)
    b = idx.shape[0]
    out = _build(b, d, 40)(idx, table)
    return out.reshape(*token_idx.shape, d)


# speedup vs baseline: 1.5823x; 1.4041x over previous
import functools
import math

import jax
import jax.numpy as jnp
from jax import lax
from jax.experimental import pallas as pl
from jax.experimental.pallas import tpu as pltpu
from jax.experimental.pallas import tpu_sc as plsc

_NC = 2
_NS = 16
_NW = _NC * _NS
_LANES = 16
_PAD = 64


@functools.lru_cache(maxsize=None)
def _build(nseq, seq_len, d):
    spw = nseq // _NW
    scale = jnp.float32(math.sqrt(d))
    mesh = plsc.VectorSubcoreMesh(core_axis_name="c", subcore_axis_name="s")
    row_buf = pltpu.VMEM((seq_len, d), jnp.float32)

    @functools.partial(
        pl.kernel,
        mesh=mesh,
        out_type=jax.ShapeDtypeStruct((nseq, seq_len, d), jnp.float32),
        scratch_types=[
            pltpu.VMEM((spw * _PAD,), jnp.int32),
            row_buf, row_buf, row_buf, row_buf,
            pltpu.SemaphoreType.DMA,
            pltpu.SemaphoreType.DMA,
            pltpu.SemaphoreType.DMA,
            pltpu.SemaphoreType.DMA,
        ],
    )
    def emb(idx_hbm, table_hbm, out_hbm, idx_v, ib0, ib1, ob0, ob1,
            si0, si1, so0, so1):
        ib = (ib0, ib1)
        ob = (ob0, ob1)
        si = (si0, si1)
        so = (so0, so1)
        wid = lax.axis_index("s") * _NC + lax.axis_index("c")
        seq_base = wid * spw
        pltpu.sync_copy(idx_hbm.at[pl.ds(seq_base * _PAD, spw * _PAD)], idx_v)

        def gather(c):
            return pltpu.async_copy(
                table_hbm.at[idx_v.at[pl.ds(c * _PAD, seq_len)]],
                ib[c % 2], si[c % 2])

        def scale_chunk(c):
            src, dst = ib[c % 2], ob[c % 2]

            def body(i, carry):
                for j in range(d // _LANES):
                    sl = pl.ds(j * _LANES, _LANES)
                    dst[i, sl] = src[i, sl] * scale
                return carry

            lax.fori_loop(0, seq_len, body, 0)

        def put(c):
            return pltpu.async_copy(
                ob[c % 2], out_hbm.at[seq_base + c], so[c % 2])

        inc = {0: gather(0)}
        if spw > 1:
            inc[1] = gather(1)
        outc = {}
        for c in range(spw):
            inc[c].wait()
            if c >= 2:
                outc[c - 2].wait()
            scale_chunk(c)
            outc[c] = put(c)
            if c + 2 < spw:
                inc[c + 2] = gather(c + 2)
        outc[spw - 2].wait()
        outc[spw - 1].wait()

    return emb


def kernel(token_idx, table):
    nseq, seq_len = token_idx.shape
    d = table.shape[1]
    idx = jnp.pad(token_idx.astype(jnp.int32), ((0, 0), (0, _PAD - seq_len)))
    return _build(nseq, seq_len, d)(idx.reshape(-1), table)
